# baseline (speedup 1.0000x reference)
import functools

import jax
import jax.numpy as jnp
from jax import lax
from jax.experimental import pallas as pl
from jax.experimental.pallas import tpu as pltpu
from jax.experimental.pallas import tpu_sc as plsc

_BM = 256
_TW = 128


def _cpn_body(x_ref, kw_ref, gw_ref, gb_ref, win_ref, tab_ref):
    K, D = kw_ref.shape
    xb = x_ref[...]
    kw = kw_ref[...]
    nrm = jnp.sqrt(jnp.sum(xb * xb, axis=1, keepdims=True))
    xn = xb / jnp.maximum(nrm, 1e-12)
    xn2 = xn + xn
    wsq = jnp.sum(kw * kw, axis=1, keepdims=True)
    s2 = lax.dot_general(
        kw, xn2, (((1,), (1,)), ((), ())),
        preferred_element_type=jnp.float32)
    negd = wsq - s2
    minv = jnp.min(negd, axis=0, keepdims=True)
    rows = lax.broadcasted_iota(jnp.int32, (K, _BM), 0)
    winners = jnp.min(jnp.where(negd == minv, rows, K),
                      axis=0, keepdims=True)
    win_ref[...] = winners[None]

    @pl.when(pl.program_id(0) == 0)
    def _():
        sig = jax.nn.sigmoid(gw_ref[...] + gb_ref[0, 0])
        tab_ref[...] = jnp.broadcast_to(sig, (K, _TW))


def _sc_gather(tab, idx):
    B = idx.shape[0]
    info = plsc.get_sparse_core_info()
    nw = info.num_cores * info.num_subcores
    b_per_w = B // nw
    mesh = plsc.VectorSubcoreMesh(core_axis_name="c", subcore_axis_name="s")

    @functools.partial(
        pl.kernel, mesh=mesh,
        out_type=jax.ShapeDtypeStruct((B, _TW), jnp.float32),
        scratch_types=[
            pltpu.VMEM((b_per_w,), jnp.int32),
            pltpu.VMEM((b_per_w, _TW), jnp.float32),
            pltpu.SemaphoreType.DMA,
        ],
    )
    def k(tab_hbm, idx_hbm, out_hbm, idx_v, rows_v, sem):
        wid = lax.axis_index("s") * info.num_cores + lax.axis_index("c")
        base = wid * b_per_w
        pltpu.sync_copy(idx_hbm.at[pl.ds(base, b_per_w)], idx_v)
        pltpu.async_copy(tab_hbm.at[idx_v], rows_v, sem).wait()
        pltpu.sync_copy(rows_v, out_hbm.at[pl.ds(base, b_per_w)])

    return k(tab, idx)


def kernel(x, kohonen_weights, grossberg_w, grossberg_b):
    B, D = x.shape
    K = kohonen_weights.shape[0]
    G = B // _BM
    gw_col = grossberg_w.reshape(K, 1)
    gb = grossberg_b.reshape(1, 1)
    win, tab = pl.pallas_call(
        _cpn_body,
        grid=(G,),
        in_specs=[
            pl.BlockSpec((_BM, D), lambda i: (i, 0)),
            pl.BlockSpec((K, D), lambda i: (0, 0)),
            pl.BlockSpec((K, 1), lambda i: (0, 0)),
            pl.BlockSpec((1, 1), lambda i: (0, 0)),
        ],
        out_specs=[
            pl.BlockSpec((1, 1, _BM), lambda i: (i, 0, 0)),
            pl.BlockSpec((K, _TW), lambda i: (0, 0)),
        ],
        out_shape=[
            jax.ShapeDtypeStruct((G, 1, _BM), jnp.int32),
            jax.ShapeDtypeStruct((K, _TW), jnp.float32),
        ],
    )(x, kohonen_weights, gw_col, gb)
    winners = win.reshape(B)
    out = _sc_gather(tab, winners)
    return out[:, :1], winners
